# baseline (speedup 1.0000x reference)
import functools

import jax
import jax.numpy as jnp
from jax import lax
from jax.experimental import pallas as pl
from jax.experimental.pallas import tpu as pltpu
from jax.experimental.pallas import tpu_sc as plsc

_LANES = 16


def _sc_counts(idx2d, n_rows, rows_out):
    num_offsets, k = idx2d.shape
    assert k % _LANES == 0 and rows_out >= num_offsets
    zeros_row = jnp.zeros((n_rows,), jnp.float32)
    mesh = plsc.VectorSubcoreMesh(core_axis_name="c", subcore_axis_name="s")

    @functools.partial(
        pl.kernel,
        out_type=jax.ShapeDtypeStruct((rows_out, n_rows), jnp.float32),
        mesh=mesh,
        scratch_types=[
            pltpu.VMEM((k,), jnp.int32),
            pltpu.VMEM((n_rows,), jnp.float32),
        ],
        compiler_params=pltpu.CompilerParams(needs_layout_passes=False),
    )
    def counts_kernel(idx_hbm, zeros_hbm, counts_hbm, idx_v, acc_v):
        wid = lax.axis_index("s") * 2 + lax.axis_index("c")

        @pl.when(wid < rows_out)
        def _():
            pltpu.sync_copy(zeros_hbm, acc_v)

            @pl.when(wid < num_offsets)
            def _():
                pltpu.sync_copy(idx_hbm.at[wid], idx_v)
                ones = jnp.full((_LANES,), 1.0, jnp.float32)
                unroll = 10
                assert k % (_LANES * unroll) == 0

                def step(i, carry):
                    base = i * (_LANES * unroll)
                    for u in range(unroll):
                        ii = idx_v[pl.ds(base + u * _LANES, _LANES)]
                        plsc.addupdate_scatter(acc_v, [ii], ones)
                    return carry

                lax.fori_loop(0, k // (_LANES * unroll), step, 0)

            pltpu.sync_copy(acc_v, counts_hbm.at[wid])

    return counts_kernel(idx2d, zeros_row)


def _tc_body(num_offsets, x_ref, ct_ref, w_ref, b_ref, o_ref):
    x = x_ref[...].astype(jnp.bfloat16)
    ct = ct_ref[...].astype(jnp.bfloat16)
    a = jnp.concatenate(
        [x * ct[:, o:o + 1] for o in range(num_offsets)], axis=1)
    y = jnp.dot(a, w_ref[...].astype(jnp.bfloat16),
                preferred_element_type=jnp.float32)
    o_ref[...] = jnp.maximum(y + b_ref[...], 0.0)


def _tc_conv(inputs, counts_t, w_flat, bias, tile_n, num_offsets):
    n, c_in = inputs.shape
    rows_out = counts_t.shape[1]
    c_out = w_flat.shape[1]
    grid = ((n + tile_n - 1) // tile_n,)
    return pl.pallas_call(
        functools.partial(_tc_body, num_offsets),
        grid=grid,
        in_specs=[
            pl.BlockSpec((tile_n, c_in), lambda i: (i, 0)),
            pl.BlockSpec((tile_n, rows_out), lambda i: (i, 0)),
            pl.BlockSpec((num_offsets * c_in, c_out), lambda i: (0, 0)),
            pl.BlockSpec((1, c_out), lambda i: (0, 0)),
        ],
        out_specs=pl.BlockSpec((tile_n, c_out), lambda i: (i, 0)),
        out_shape=jax.ShapeDtypeStruct((n, c_out), jnp.float32),
        compiler_params=pltpu.CompilerParams(
            dimension_semantics=("parallel",),
        ),
    )(inputs, counts_t, w_flat, bias)


def kernel(inputs, voxel_idx, weight_idx_to_input_idxs, kernel, bias):
    n, c_in = inputs.shape
    num_offsets, k, _ = weight_idx_to_input_idxs.shape
    c_out = kernel.shape[-1]

    idx2d = weight_idx_to_input_idxs.reshape(num_offsets, k)
    rows_out = (num_offsets + 7) // 8 * 8
    counts = _sc_counts(idx2d, n, rows_out)
    counts_t = counts.T
    w_flat = kernel.reshape(num_offsets * c_in, c_out)
    return _tc_conv(inputs, counts_t, w_flat, bias, tile_n=4096,
                    num_offsets=num_offsets)


# speedup vs baseline: 2.0019x; 2.0019x over previous
import functools

import jax
import jax.numpy as jnp
from jax import lax
from jax.experimental import pallas as pl
from jax.experimental.pallas import tpu as pltpu
from jax.experimental.pallas import tpu_sc as plsc

_LANES = 16


def _sc_counts(idx2d, n_rows, rows_out):
    num_offsets, k = idx2d.shape
    assert k % _LANES == 0 and rows_out >= num_offsets
    zeros_row = jnp.zeros((n_rows,), jnp.float32)
    mesh = plsc.VectorSubcoreMesh(core_axis_name="c", subcore_axis_name="s")

    @functools.partial(
        pl.kernel,
        out_type=jax.ShapeDtypeStruct((rows_out, n_rows), jnp.float32),
        mesh=mesh,
        scratch_types=[
            pltpu.VMEM((k,), jnp.int32),
            pltpu.VMEM((n_rows,), jnp.float32),
        ],
        compiler_params=pltpu.CompilerParams(needs_layout_passes=False),
    )
    def counts_kernel(idx_hbm, zeros_hbm, counts_hbm, idx_v, acc_v):
        wid = lax.axis_index("s") * 2 + lax.axis_index("c")

        @pl.when(wid < rows_out)
        def _():
            pltpu.sync_copy(zeros_hbm, acc_v)

            @pl.when(wid < num_offsets)
            def _():
                pltpu.sync_copy(idx_hbm.at[wid], idx_v)
                ones = jnp.full((_LANES,), 1.0, jnp.float32)
                unroll = 10
                assert k % (_LANES * unroll) == 0

                def step(i, carry):
                    base = i * (_LANES * unroll)
                    for u in range(unroll):
                        ii = idx_v[pl.ds(base + u * _LANES, _LANES)]
                        plsc.addupdate_scatter(acc_v, [ii], ones)
                    return carry

                lax.fori_loop(0, k // (_LANES * unroll), step, 0)

            pltpu.sync_copy(acc_v, counts_hbm.at[wid])

    return counts_kernel(idx2d, zeros_row)


def _tc_body(num_offsets, x_ref, c_ref, w_ref, b_ref, o_ref):
    xt = x_ref[...].astype(jnp.bfloat16).T
    c = c_ref[...].astype(jnp.bfloat16)
    at = jnp.concatenate(
        [xt * c[o:o + 1, :] for o in range(num_offsets)], axis=0)
    yt = lax.dot_general(w_ref[...].astype(jnp.bfloat16), at,
                         (((0,), (0,)), ((), ())),
                         preferred_element_type=jnp.float32)
    o_ref[...] = jnp.maximum(yt.T + b_ref[...], 0.0)


def _tc_conv(inputs, counts, w_flat, bias, tile_n, num_offsets):
    n, c_in = inputs.shape
    rows_out = counts.shape[0]
    c_out = w_flat.shape[1]
    grid = ((n + tile_n - 1) // tile_n,)
    return pl.pallas_call(
        functools.partial(_tc_body, num_offsets),
        grid=grid,
        in_specs=[
            pl.BlockSpec((tile_n, c_in), lambda i: (i, 0)),
            pl.BlockSpec((rows_out, tile_n), lambda i: (0, i)),
            pl.BlockSpec((num_offsets * c_in, c_out), lambda i: (0, 0)),
            pl.BlockSpec((1, c_out), lambda i: (0, 0)),
        ],
        out_specs=pl.BlockSpec((tile_n, c_out), lambda i: (i, 0)),
        out_shape=jax.ShapeDtypeStruct((n, c_out), jnp.float32),
        compiler_params=pltpu.CompilerParams(
            dimension_semantics=("parallel",),
        ),
    )(inputs, counts, w_flat, bias)


def kernel(inputs, voxel_idx, weight_idx_to_input_idxs, kernel, bias):
    n, c_in = inputs.shape
    num_offsets, k, _ = weight_idx_to_input_idxs.shape
    c_out = kernel.shape[-1]

    idx2d = weight_idx_to_input_idxs.reshape(num_offsets, k)
    rows_out = (num_offsets + 7) // 8 * 8
    counts = _sc_counts(idx2d, n, rows_out)
    w_flat = kernel.reshape(num_offsets * c_in, c_out)
    return _tc_conv(inputs, counts, w_flat, bias, tile_n=8192,
                    num_offsets=num_offsets)


# speedup vs baseline: 2.0258x; 1.0120x over previous
import functools

import jax
import jax.numpy as jnp
from jax import lax
from jax.experimental import pallas as pl
from jax.experimental.pallas import tpu as pltpu
from jax.experimental.pallas import tpu_sc as plsc

_LANES = 16


def _sc_counts(idx2d, n_rows, rows_out):
    num_offsets, k = idx2d.shape
    assert k % _LANES == 0 and rows_out >= num_offsets
    zeros_row = jnp.zeros((n_rows,), jnp.float32)
    mesh = plsc.VectorSubcoreMesh(core_axis_name="c", subcore_axis_name="s")

    @functools.partial(
        pl.kernel,
        out_type=jax.ShapeDtypeStruct((rows_out, n_rows), jnp.float32),
        mesh=mesh,
        scratch_types=[
            pltpu.VMEM((k,), jnp.int32),
            pltpu.VMEM((n_rows,), jnp.float32),
        ],
        compiler_params=pltpu.CompilerParams(needs_layout_passes=False),
    )
    def counts_kernel(idx_hbm, zeros_hbm, counts_hbm, idx_v, acc_v):
        wid = lax.axis_index("s") * 2 + lax.axis_index("c")

        @pl.when(wid < rows_out)
        def _():
            pltpu.sync_copy(zeros_hbm, acc_v)

            @pl.when(wid < num_offsets)
            def _():
                pltpu.sync_copy(idx_hbm.at[wid], idx_v)
                ones = jnp.full((_LANES,), 1.0, jnp.float32)
                unroll = 10
                assert k % (_LANES * unroll) == 0

                def step(i, carry):
                    base = i * (_LANES * unroll)
                    for u in range(unroll):
                        ii = idx_v[pl.ds(base + u * _LANES, _LANES)]
                        plsc.addupdate_scatter(acc_v, [ii], ones)
                    return carry

                lax.fori_loop(0, k // (_LANES * unroll), step, 0)

            pltpu.sync_copy(acc_v, counts_hbm.at[wid])

    return counts_kernel(idx2d, zeros_row)


def _tc_body(num_offsets, x_ref, c_ref, w_ref, b_ref, o_ref):
    xt = x_ref[...].astype(jnp.bfloat16).T
    c = c_ref[...].astype(jnp.bfloat16)
    at = jnp.concatenate(
        [xt * c[o:o + 1, :] for o in range(num_offsets)], axis=0)
    yt = lax.dot_general(w_ref[...].astype(jnp.bfloat16), at,
                         (((0,), (0,)), ((), ())),
                         preferred_element_type=jnp.float32)
    o_ref[...] = jnp.maximum(yt.T + b_ref[...], 0.0)


def _tc_conv(inputs, counts, w_flat, bias, tile_n, num_offsets):
    n, c_in = inputs.shape
    rows_out = counts.shape[0]
    c_out = w_flat.shape[1]
    grid = ((n + tile_n - 1) // tile_n,)
    return pl.pallas_call(
        functools.partial(_tc_body, num_offsets),
        grid=grid,
        in_specs=[
            pl.BlockSpec((tile_n, c_in), lambda i: (i, 0)),
            pl.BlockSpec((rows_out, tile_n), lambda i: (0, i)),
            pl.BlockSpec((num_offsets * c_in, c_out), lambda i: (0, 0)),
            pl.BlockSpec((1, c_out), lambda i: (0, 0)),
        ],
        out_specs=pl.BlockSpec((tile_n, c_out), lambda i: (i, 0)),
        out_shape=jax.ShapeDtypeStruct((n, c_out), jnp.float32),
        compiler_params=pltpu.CompilerParams(
            dimension_semantics=("parallel",),
        ),
    )(inputs, counts, w_flat, bias)


def kernel(inputs, voxel_idx, weight_idx_to_input_idxs, kernel, bias):
    n, c_in = inputs.shape
    num_offsets, k, _ = weight_idx_to_input_idxs.shape
    c_out = kernel.shape[-1]

    idx2d = weight_idx_to_input_idxs.reshape(num_offsets, k)
    rows_out = (num_offsets + 7) // 8 * 8
    counts = _sc_counts(idx2d, n, rows_out)
    w_flat = kernel.reshape(num_offsets * c_in, c_out)
    return _tc_conv(inputs, counts, w_flat, bias, tile_n=2048,
                    num_offsets=num_offsets)


# speedup vs baseline: 2.2126x; 1.0922x over previous
import functools

import jax
import jax.numpy as jnp
from jax import lax
from jax.experimental import pallas as pl
from jax.experimental.pallas import tpu as pltpu
from jax.experimental.pallas import tpu_sc as plsc

_LANES = 16


def _sc_counts(idx2d, n_rows):
    num_offsets, k = idx2d.shape
    assert k % _LANES == 0
    mesh = plsc.VectorSubcoreMesh(core_axis_name="c", subcore_axis_name="s")

    @functools.partial(
        pl.kernel,
        out_type=jax.ShapeDtypeStruct((num_offsets, n_rows), jnp.float32),
        mesh=mesh,
        scratch_types=[
            pltpu.VMEM((k,), jnp.int32),
            pltpu.VMEM((n_rows,), jnp.float32),
            pltpu.SemaphoreType.DMA,
        ],
        compiler_params=pltpu.CompilerParams(needs_layout_passes=False),
    )
    def counts_kernel(idx_hbm, counts_hbm, idx_v, acc_v, sem):
        wid = lax.axis_index("s") * 2 + lax.axis_index("c")

        @pl.when(wid < num_offsets)
        def _():
            copy = pltpu.async_copy(idx_hbm.at[wid], idx_v, sem)
            zv = jnp.zeros((_LANES,), jnp.float32)
            zunroll = 5
            assert n_rows % (_LANES * zunroll) == 0

            def zstep(i, carry):
                base = i * (_LANES * zunroll)
                for u in range(zunroll):
                    acc_v[pl.ds(base + u * _LANES, _LANES)] = zv
                return carry

            lax.fori_loop(0, n_rows // (_LANES * zunroll), zstep, 0)
            copy.wait()

            ones = jnp.full((_LANES,), 1.0, jnp.float32)
            unroll = 10
            assert k % (_LANES * unroll) == 0

            def step(i, carry):
                base = i * (_LANES * unroll)
                for u in range(unroll):
                    ii = idx_v[pl.ds(base + u * _LANES, _LANES)]
                    plsc.addupdate_scatter(acc_v, [ii], ones)
                return carry

            lax.fori_loop(0, k // (_LANES * unroll), step, 0)
            pltpu.sync_copy(acc_v, counts_hbm.at[wid])

    return counts_kernel(idx2d)


def _tc_body(num_offsets, x_ref, c_ref, w_ref, b_ref, o_ref):
    xt = x_ref[...].astype(jnp.bfloat16).T
    c = c_ref[...].astype(jnp.bfloat16)
    at = jnp.concatenate(
        [xt * c[o:o + 1, :] for o in range(num_offsets)], axis=0)
    yt = lax.dot_general(w_ref[...].astype(jnp.bfloat16), at,
                         (((0,), (0,)), ((), ())),
                         preferred_element_type=jnp.float32)
    o_ref[...] = jnp.maximum(yt.T + b_ref[...], 0.0)


def _tc_conv(inputs, counts, w_flat, bias, tile_n, num_offsets):
    n, c_in = inputs.shape
    rows_out = counts.shape[0]
    c_out = w_flat.shape[1]
    grid = ((n + tile_n - 1) // tile_n,)
    return pl.pallas_call(
        functools.partial(_tc_body, num_offsets),
        grid=grid,
        in_specs=[
            pl.BlockSpec((tile_n, c_in), lambda i: (i, 0)),
            pl.BlockSpec((rows_out, tile_n), lambda i: (0, i)),
            pl.BlockSpec((num_offsets * c_in, c_out), lambda i: (0, 0)),
            pl.BlockSpec((1, c_out), lambda i: (0, 0)),
        ],
        out_specs=pl.BlockSpec((tile_n, c_out), lambda i: (i, 0)),
        out_shape=jax.ShapeDtypeStruct((n, c_out), jnp.float32),
        compiler_params=pltpu.CompilerParams(
            dimension_semantics=("parallel",),
        ),
    )(inputs, counts, w_flat, bias)


def kernel(inputs, voxel_idx, weight_idx_to_input_idxs, kernel, bias):
    n, c_in = inputs.shape
    num_offsets, k, _ = weight_idx_to_input_idxs.shape
    c_out = kernel.shape[-1]

    idx2d = weight_idx_to_input_idxs.reshape(num_offsets, k)
    counts = _sc_counts(idx2d, n)
    w_flat = kernel.reshape(num_offsets * c_in, c_out)
    return _tc_conv(inputs, counts, w_flat, bias, tile_n=4096,
                    num_offsets=num_offsets)
